# baseline (speedup 1.0000x reference)
import functools

import jax
import jax.numpy as jnp
from jax import lax
from jax.experimental import pallas as pl
from jax.experimental.pallas import tpu as pltpu
from jax.experimental.pallas import tpu_sc as plsc

_NC = 2
_NS = 16
_NW = _NC * _NS
_CHUNK = 16384


def _sc_body(n_elems, fp_hbm, x_hbm, out_hbm, fpv, inbuf, outbuf,
             sem_in0, sem_in1, sem_out0, sem_out1):
    wid = lax.axis_index("s") * _NC + lax.axis_index("c")
    per_w = n_elems // _NW
    steps = per_w // _CHUNK
    base = wid * per_w

    pltpu.sync_copy(fp_hbm, fpv)
    a0 = fpv[0, :]
    a1 = fpv[1, :]
    a2 = fpv[2, :]
    a3 = fpv[3, :]
    b0, b1 = jnp.minimum(a0, a1), jnp.maximum(a0, a1)
    b2, b3 = jnp.minimum(a2, a3), jnp.maximum(a2, a3)
    c0, c2 = jnp.minimum(b0, b2), jnp.maximum(b0, b2)
    c1, c3 = jnp.minimum(b1, b3), jnp.maximum(b1, b3)
    v0, v3 = c0, c3
    v1, v2 = jnp.minimum(c1, c2), jnp.maximum(c1, c2)
    m1 = (v0 + v1) * 0.5
    m2 = (v1 + v2) * 0.5
    m3 = (v2 + v3) * 0.5

    sems_in = (sem_in0, sem_in1)
    sems_out = (sem_out0, sem_out1)

    def in_copy(g, slot):
        return pltpu.make_async_copy(
            x_hbm.at[pl.ds(base + g * _CHUNK, _CHUNK)], inbuf.at[slot],
            sems_in[slot])

    def out_copy(g, slot):
        return pltpu.make_async_copy(
            outbuf.at[slot], out_hbm.at[pl.ds(base + g * _CHUNK, _CHUNK)],
            sems_out[slot])

    in_copy(0, 0).start()
    in_copy(1, 1).start()

    def step(g, slot):
        @pl.when(g >= 2)
        def _():
            out_copy(g - 2, slot).wait()
        in_copy(g, slot).wait()
        src = inbuf.at[slot]
        dst = outbuf.at[slot]

        def body(i, _):
            b = i * 64
            for u in range(4):
                xv = src[pl.ds(b + u * 16, 16)]
                r = jnp.where(xv > m2,
                              jnp.where(xv > m3, v3, v2),
                              jnp.where(xv > m1, v1, v0))
                dst[pl.ds(b + u * 16, 16)] = r
            return 0

        lax.fori_loop(0, _CHUNK // 64, body, 0)
        out_copy(g, slot).start()

        @pl.when(g + 2 < steps)
        def _():
            in_copy(g + 2, slot).start()

    def pair(p, _):
        g = p * 2
        step(g, 0)
        step(g + 1, 1)
        return 0

    lax.fori_loop(0, steps // 2, pair, 0)
    out_copy(steps - 2, 0).wait()
    out_copy(steps - 1, 1).wait()


def _sc_quant(fp_pad, x_flat):
    n = x_flat.shape[0]
    mesh = plsc.VectorSubcoreMesh(core_axis_name="c", subcore_axis_name="s")
    return pl.kernel(
        functools.partial(_sc_body, n),
        out_type=jax.ShapeDtypeStruct((n,), jnp.float32),
        mesh=mesh,
        scratch_types=[
            pltpu.VMEM((4, 16), jnp.float32),
            pltpu.VMEM((2, _CHUNK), jnp.float32),
            pltpu.VMEM((2, _CHUNK), jnp.float32),
            pltpu.SemaphoreType.DMA,
            pltpu.SemaphoreType.DMA,
            pltpu.SemaphoreType.DMA,
            pltpu.SemaphoreType.DMA,
        ],
    )(fp_pad, x_flat)


def kernel(x, fp_values):
    n = x.size
    fp_pad = jnp.broadcast_to(fp_values.reshape(4, 1), (4, 16))
    fp_pad = jnp.asarray(fp_pad, jnp.float32)
    out = _sc_quant(fp_pad, x.reshape(n))
    return out.reshape(x.shape)


# speedup vs baseline: 1.0017x; 1.0017x over previous
import functools

import jax
import jax.numpy as jnp
from jax import lax
from jax.experimental import pallas as pl
from jax.experimental.pallas import tpu as pltpu
from jax.experimental.pallas import tpu_sc as plsc

_NC = 2
_NS = 16
_NW = _NC * _NS
_CHUNK = 16384


def _sc_body(n_elems, fp_hbm, x_hbm, out_hbm, fpv, inbuf, outbuf,
             sem_in0, sem_in1, sem_out0, sem_out1):
    wid = lax.axis_index("s") * _NC + lax.axis_index("c")
    per_w = n_elems // _NW
    steps = per_w // _CHUNK
    base = wid * per_w

    pltpu.sync_copy(fp_hbm, fpv)
    a0 = fpv[0, :]
    a1 = fpv[1, :]
    a2 = fpv[2, :]
    a3 = fpv[3, :]
    b0, b1 = jnp.minimum(a0, a1), jnp.maximum(a0, a1)
    b2, b3 = jnp.minimum(a2, a3), jnp.maximum(a2, a3)
    c0, c2 = jnp.minimum(b0, b2), jnp.maximum(b0, b2)
    c1, c3 = jnp.minimum(b1, b3), jnp.maximum(b1, b3)
    v0, v3 = c0, c3
    v1, v2 = jnp.minimum(c1, c2), jnp.maximum(c1, c2)
    m1 = (v0 + v1) * 0.5
    m2 = (v1 + v2) * 0.5
    m3 = (v2 + v3) * 0.5

    sems_in = (sem_in0, sem_in1)
    sems_out = (sem_out0, sem_out1)

    def in_copy(g, slot):
        return pltpu.make_async_copy(
            x_hbm.at[pl.ds(base + g * _CHUNK, _CHUNK)], inbuf.at[slot],
            sems_in[slot])

    def out_copy(g, slot):
        return pltpu.make_async_copy(
            outbuf.at[slot], out_hbm.at[pl.ds(base + g * _CHUNK, _CHUNK)],
            sems_out[slot])

    in_copy(0, 0).start()
    in_copy(1, 1).start()

    def step(g, slot):
        @pl.when(g >= 2)
        def _():
            out_copy(g - 2, slot).wait()
        in_copy(g, slot).wait()
        src = inbuf.at[slot]
        dst = outbuf.at[slot]

        @plsc.parallel_loop(0, _CHUNK, step=16, unroll=8)
        def body(i):
            xv = src[pl.ds(i, 16)]
            r = jnp.where(xv > m2,
                          jnp.where(xv > m3, v3, v2),
                          jnp.where(xv > m1, v1, v0))
            dst[pl.ds(i, 16)] = r
        out_copy(g, slot).start()

        @pl.when(g + 2 < steps)
        def _():
            in_copy(g + 2, slot).start()

    def pair(p, _):
        g = p * 2
        step(g, 0)
        step(g + 1, 1)
        return 0

    lax.fori_loop(0, steps // 2, pair, 0)
    out_copy(steps - 2, 0).wait()
    out_copy(steps - 1, 1).wait()


def _sc_quant(fp_pad, x_flat):
    n = x_flat.shape[0]
    mesh = plsc.VectorSubcoreMesh(core_axis_name="c", subcore_axis_name="s")
    return pl.kernel(
        functools.partial(_sc_body, n),
        out_type=jax.ShapeDtypeStruct((n,), jnp.float32),
        mesh=mesh,
        scratch_types=[
            pltpu.VMEM((4, 16), jnp.float32),
            pltpu.VMEM((2, _CHUNK), jnp.float32),
            pltpu.VMEM((2, _CHUNK), jnp.float32),
            pltpu.SemaphoreType.DMA,
            pltpu.SemaphoreType.DMA,
            pltpu.SemaphoreType.DMA,
            pltpu.SemaphoreType.DMA,
        ],
    )(fp_pad, x_flat)


def kernel(x, fp_values):
    n = x.size
    fp_pad = jnp.broadcast_to(fp_values.reshape(4, 1), (4, 16))
    fp_pad = jnp.asarray(fp_pad, jnp.float32)
    out = _sc_quant(fp_pad, x.reshape(n))
    return out.reshape(x.shape)


# speedup vs baseline: 3.3380x; 3.3325x over previous
import functools

import jax
import jax.numpy as jnp
from jax import lax
from jax.experimental import pallas as pl
from jax.experimental.pallas import tpu as pltpu
from jax.experimental.pallas import tpu_sc as plsc

_NC = 2
_NS = 16
_NW = _NC * _NS
_CROWS = 8


def _sc_body(rows, cols, fp_hbm, x_hbm, out_hbm, fpv, inbuf, outbuf,
             sem_in0, sem_in1, sem_out0, sem_out1):
    wid = lax.axis_index("s") * _NC + lax.axis_index("c")
    per_w = rows // _NW
    steps = per_w // _CROWS
    base = wid * per_w

    pltpu.sync_copy(fp_hbm, fpv)
    a0 = fpv[0, :]
    a1 = fpv[1, :]
    a2 = fpv[2, :]
    a3 = fpv[3, :]
    b0, b1 = jnp.minimum(a0, a1), jnp.maximum(a0, a1)
    b2, b3 = jnp.minimum(a2, a3), jnp.maximum(a2, a3)
    c0, c2 = jnp.minimum(b0, b2), jnp.maximum(b0, b2)
    c1, c3 = jnp.minimum(b1, b3), jnp.maximum(b1, b3)
    v0, v3 = c0, c3
    v1, v2 = jnp.minimum(c1, c2), jnp.maximum(c1, c2)
    m1 = (v0 + v1) * 0.5
    m2 = (v1 + v2) * 0.5
    m3 = (v2 + v3) * 0.5

    sems_in = (sem_in0, sem_in1)
    sems_out = (sem_out0, sem_out1)

    def in_copy(g, slot):
        return pltpu.make_async_copy(
            x_hbm.at[pl.ds(base + g * _CROWS, _CROWS)], inbuf.at[slot],
            sems_in[slot])

    def out_copy(g, slot):
        return pltpu.make_async_copy(
            outbuf.at[slot], out_hbm.at[pl.ds(base + g * _CROWS, _CROWS)],
            sems_out[slot])

    in_copy(0, 0).start()
    in_copy(1, 1).start()

    def step(g, slot):
        @pl.when(g >= 2)
        def _():
            out_copy(g - 2, slot).wait()
        in_copy(g, slot).wait()
        src = inbuf.at[slot]
        dst = outbuf.at[slot]

        for r in range(_CROWS):
            @plsc.parallel_loop(0, cols, step=16, unroll=8)
            def body(i):
                xv = src[r, pl.ds(i, 16)]
                q = jnp.where(xv > m2,
                              jnp.where(xv > m3, v3, v2),
                              jnp.where(xv > m1, v1, v0))
                dst[r, pl.ds(i, 16)] = q

        out_copy(g, slot).start()

        @pl.when(g + 2 < steps)
        def _():
            in_copy(g + 2, slot).start()

    def pair(p, _):
        g = p * 2
        step(g, 0)
        step(g + 1, 1)
        return 0

    lax.fori_loop(0, steps // 2, pair, 0)
    out_copy(steps - 2, 0).wait()
    out_copy(steps - 1, 1).wait()


def _sc_quant(fp_bcast, x2):
    rows, cols = x2.shape
    mesh = plsc.VectorSubcoreMesh(core_axis_name="c", subcore_axis_name="s")
    return pl.kernel(
        functools.partial(_sc_body, rows, cols),
        out_type=jax.ShapeDtypeStruct((rows, cols), jnp.float32),
        mesh=mesh,
        scratch_types=[
            pltpu.VMEM((4, 16), jnp.float32),
            pltpu.VMEM((2, _CROWS, 2048), jnp.float32),
            pltpu.VMEM((2, _CROWS, 2048), jnp.float32),
            pltpu.SemaphoreType.DMA,
            pltpu.SemaphoreType.DMA,
            pltpu.SemaphoreType.DMA,
            pltpu.SemaphoreType.DMA,
        ],
    )(fp_bcast, x2)


def kernel(x, fp_values):
    fp_bcast = jnp.asarray(
        jnp.broadcast_to(fp_values.reshape(4, 1), (4, 16)), jnp.float32)
    x2 = x.reshape(-1, x.shape[-1])
    out = _sc_quant(fp_bcast, x2)
    return out.reshape(x.shape)
